# baseline (speedup 1.0000x reference)
import functools

import jax
import jax.numpy as jnp
from jax import lax
from jax.experimental import pallas as pl
from jax.experimental.pallas import tpu as pltpu
from jax.experimental.pallas import tpu_sc as plsc

_NS = 16
_NC = 2
_NW = _NC * _NS
_CK = 128


def _cdiv(a, b):
    return (a + b - 1) // b


def _deg_call(dst1d, npad, cpt):
    mesh = plsc.VectorSubcoreMesh(core_axis_name="c", subcore_axis_name="s")

    @functools.partial(
        pl.kernel,
        mesh=mesh,
        out_type=jax.ShapeDtypeStruct((_NW * npad,), jnp.float32),
        compiler_params=pltpu.CompilerParams(needs_layout_passes=False),
        scratch_types=[
            pltpu.VMEM((_CK,), jnp.int32),
            pltpu.VMEM((npad,), jnp.float32),
        ],
    )
    def k(dst_hbm, out_hbm, didx, priv):
        c = lax.axis_index("c")
        s = lax.axis_index("s")
        wid = c * _NS + s

        def zero(i, _):
            priv[pl.ds(i * 16, 16)] = jnp.zeros((16,), jnp.float32)
            return 0

        lax.fori_loop(0, npad // 16, zero, 0)
        ones16 = jnp.ones((16,), jnp.float32)

        def body(j, _):
            off = (wid * cpt + j) * _CK
            pltpu.sync_copy(dst_hbm.at[pl.ds(off, _CK)], didx)
            for i in range(_CK // 16):
                idx = didx[pl.ds(i * 16, 16)]
                plsc.addupdate_scatter(priv, [idx], ones16)
            return 0

        lax.fori_loop(0, cpt, body, 0)
        pltpu.sync_copy(priv, out_hbm.at[pl.ds(wid * npad, npad)])

    return k(dst1d)


_CKL = 128


def _gs_call(g, src1d, dst1d, cpt0, cpt1):
    n, h = g.shape
    mesh = plsc.VectorSubcoreMesh(core_axis_name="c", subcore_axis_name="s")

    @functools.partial(
        pl.kernel,
        mesh=mesh,
        out_type=(),
        scratch_types=[
            pltpu.VMEM((_CKL,), jnp.int32),
            pltpu.VMEM((_CKL,), jnp.int32),
            pltpu.VMEM((_CKL, h), jnp.float32),
            pltpu.SemaphoreType.DMA,
            pltpu.SemaphoreType.DMA,
        ],
    )
    def k(g_hbm, src_hbm, dst_hbm, acc_ref, sidx, didx, rows, gsem, ssem):
        c = lax.axis_index("c")
        s = lax.axis_index("s")
        base = jnp.where(c == 0, s * cpt0,
                         _NS * cpt0 + s * cpt1) * _CKL
        nch = jnp.where(c == 0, cpt0, cpt1)

        def body(j, _):
            off = base + j * _CKL
            pltpu.sync_copy(src_hbm.at[pl.ds(off, _CKL)], sidx)
            pltpu.sync_copy(dst_hbm.at[pl.ds(off, _CKL)], didx)
            pltpu.async_copy(g_hbm.at[sidx], rows, gsem).wait()
            pltpu.async_copy(rows, acc_ref.at[didx], ssem, add=True).wait()
            return 0

        lax.fori_loop(0, nch, body, 0)

    acc = jax.new_ref(jnp.concatenate([g, jnp.zeros((8, h), jnp.float32)]))
    k(g, src1d, dst1d, acc)
    return acc[...][:n]


def _dinv_call(parts):
    def body(p_ref, o_ref):
        deg = jnp.sum(p_ref[...], axis=0, keepdims=True)
        o_ref[...] = lax.rsqrt(deg + 1.0)

    return pl.pallas_call(
        body,
        out_shape=jax.ShapeDtypeStruct((1, parts.shape[1]), jnp.float32),
    )(parts)


def _mm_call(dinv, a, g, W, b):
    n, d = a.shape
    h = W.shape[1]
    bm = 2000 if n % 2000 == 0 else n
    grid = n // bm

    def body(dv_ref, a_ref, W_ref, b_ref, o_ref):
        av = a_ref[...]
        dv = dv_ref[...]
        if b is not None:
            av = dv * jnp.maximum(dv * av + b_ref[...], 0.0)
        else:
            av = dv * av
        o_ref[...] = lax.dot_general(av, W_ref[...], (((1,), (0,)), ((), ())),
                                     preferred_element_type=jnp.float32)

    bias = jnp.zeros((1, d), jnp.float32) if b is None else b.reshape(1, d)
    del g
    return pl.pallas_call(
        body,
        grid=(grid,),
        in_specs=[
            pl.BlockSpec((bm, 1), lambda i: (i, 0)),
            pl.BlockSpec((bm, d), lambda i: (i, 0)),
            pl.BlockSpec((d, h), lambda i: (0, 0)),
            pl.BlockSpec((1, d), lambda i: (0, 0)),
        ],
        out_specs=pl.BlockSpec((bm, h), lambda i: (i, 0)),
        out_shape=jax.ShapeDtypeStruct((n, h), jnp.float32),
    )(dinv, a, W, bias)


def _logits_call(dinv, a, Wa, b, ba):
    n, h = a.shape
    bm = 2000 if n % 2000 == 0 else n
    grid = n // bm

    def body(dv_ref, a_ref, Wa_ref, b_ref, ba_ref, o_ref):
        hv = jnp.maximum(dv_ref[...] * a_ref[...] + b_ref[...], 0.0)
        o_ref[...] = lax.dot_general(hv, Wa_ref[...], (((1,), (0,)), ((), ())),
                                     preferred_element_type=jnp.float32) + ba_ref[...]

    return pl.pallas_call(
        body,
        grid=(grid,),
        in_specs=[
            pl.BlockSpec((bm, 1), lambda i: (i, 0)),
            pl.BlockSpec((bm, h), lambda i: (i, 0)),
            pl.BlockSpec((h, 1), lambda i: (0, 0)),
            pl.BlockSpec((1, h), lambda i: (0, 0)),
            pl.BlockSpec((1, 1), lambda i: (0, 0)),
        ],
        out_specs=pl.BlockSpec((bm, 1), lambda i: (i, 0)),
        out_shape=jax.ShapeDtypeStruct((n, 1), jnp.float32),
    )(dinv, a, Wa, b.reshape(1, h), ba.reshape(1, 1))


def _softmax_call(logits, batch, maskf, num_seg):
    n = logits.shape[0]
    neg = -1e30

    def body(lg_ref, bt_ref, m_ref, o_ref):
        lg = lg_ref[...]
        bt = bt_ref[...]
        m = m_ref[...]
        masked = jnp.where(m > 0, lg, neg)
        seg = lax.broadcasted_iota(jnp.int32, (1, num_seg), 1)
        onehot = bt == seg
        vals = jnp.where(onehot, masked, neg)
        smax = jnp.max(vals, axis=0, keepdims=True)
        smax = jnp.where(smax > -1e29, smax, 0.0)
        pmax = jnp.sum(jnp.where(onehot, smax, 0.0), axis=1, keepdims=True)
        ex = jnp.where(m > 0, jnp.exp(masked - pmax), 0.0)
        den = jnp.sum(jnp.where(onehot, ex, 0.0), axis=0, keepdims=True)
        pden = jnp.sum(jnp.where(onehot, den, 0.0), axis=1, keepdims=True)
        o_ref[...] = ex / pden

    return pl.pallas_call(
        body,
        out_shape=jax.ShapeDtypeStruct((n, 1), jnp.float32),
    )(logits, batch, maskf)


def kernel(x, edge_index, mask, batch, W1, b1, W2, b2, W3, b3, Wa, ba):
    n, d = x.shape
    e = edge_index.shape[1]
    src = edge_index[0]
    dst = edge_index[1]

    t_pair = 2 * _cdiv(e, _NW * _CK)
    cpt0 = (3 * t_pair) // 8
    cpt1 = t_pair - cpt0
    e_pad = _NS * t_pair * _CK
    cpt = t_pair // 2
    src_p = jnp.concatenate([src, jnp.zeros((e_pad - e,), jnp.int32)])
    pad_i = jnp.arange(e, e_pad, dtype=jnp.int32)
    dst_p = jnp.concatenate([dst, n + (pad_i % 8)])
    npad = 16 * _cdiv(n + 1, 16)

    parts = _deg_call(dst_p, npad, cpt).reshape(_NW, npad)
    dinv = _dinv_call(parts).reshape(npad, 1)[:n]

    g1 = _mm_call(dinv, x, None, W1, None)
    a1 = _gs_call(g1, src_p, dst_p, cpt0, cpt1)
    g2 = _mm_call(dinv, a1, g1, W2, b1)
    a2 = _gs_call(g2, src_p, dst_p, cpt0, cpt1)
    g3 = _mm_call(dinv, a2, g2, W3, b2)
    a3 = _gs_call(g3, src_p, dst_p, cpt0, cpt1)

    logits = _logits_call(dinv, a3, Wa, b3, ba)
    probs = _softmax_call(logits, batch.reshape(n, 1),
                          mask.astype(jnp.float32).reshape(n, 1), 32)
    return probs.reshape(n)


# speedup vs baseline: 1.1290x; 1.1290x over previous
import functools

import jax
import jax.numpy as jnp
from jax import lax
from jax.experimental import pallas as pl
from jax.experimental.pallas import tpu as pltpu
from jax.experimental.pallas import tpu_sc as plsc

_NS = 16
_NC = 2
_NW = _NC * _NS
_CK = 128


def _cdiv(a, b):
    return (a + b - 1) // b


def _deg_call(dst1d, npad, cpt):
    mesh = plsc.VectorSubcoreMesh(core_axis_name="c", subcore_axis_name="s")

    @functools.partial(
        pl.kernel,
        mesh=mesh,
        out_type=jax.ShapeDtypeStruct((_NW * npad,), jnp.float32),
        compiler_params=pltpu.CompilerParams(needs_layout_passes=False),
        scratch_types=[
            pltpu.VMEM((_CK,), jnp.int32),
            pltpu.VMEM((npad,), jnp.float32),
        ],
    )
    def k(dst_hbm, out_hbm, didx, priv):
        c = lax.axis_index("c")
        s = lax.axis_index("s")
        wid = c * _NS + s

        def zero(i, _):
            priv[pl.ds(i * 16, 16)] = jnp.zeros((16,), jnp.float32)
            return 0

        lax.fori_loop(0, npad // 16, zero, 0)
        ones16 = jnp.ones((16,), jnp.float32)

        def body(j, _):
            off = (wid * cpt + j) * _CK
            pltpu.sync_copy(dst_hbm.at[pl.ds(off, _CK)], didx)
            for i in range(_CK // 16):
                idx = didx[pl.ds(i * 16, 16)]
                plsc.addupdate_scatter(priv, [idx], ones16)
            return 0

        lax.fori_loop(0, cpt, body, 0)
        pltpu.sync_copy(priv, out_hbm.at[pl.ds(wid * npad, npad)])

    return k(dst1d)


_CKL = 128


def _gs_call(g, src1d, dst1d, cpt0, cpt1):
    n, h = g.shape
    mesh = plsc.VectorSubcoreMesh(core_axis_name="c", subcore_axis_name="s")

    @functools.partial(
        pl.kernel,
        mesh=mesh,
        out_type=(),
        scratch_types=[
            pltpu.VMEM((_CKL,), jnp.int32),
            pltpu.VMEM((_CKL,), jnp.int32),
            pltpu.VMEM((_CKL, h), jnp.float32),
            pltpu.SemaphoreType.DMA,
            pltpu.SemaphoreType.DMA,
        ],
    )
    def k(g_hbm, src_hbm, dst_hbm, acc_ref, sidx, didx, rows, gsem, ssem):
        c = lax.axis_index("c")
        s = lax.axis_index("s")
        base = jnp.where(c == 0, s * cpt0,
                         _NS * cpt0 + s * cpt1) * _CKL
        nch = jnp.where(c == 0, cpt0, cpt1)

        def body(j, _):
            off = base + j * _CKL
            pltpu.sync_copy(src_hbm.at[pl.ds(off, _CKL)], sidx)
            pltpu.sync_copy(dst_hbm.at[pl.ds(off, _CKL)], didx)
            pltpu.async_copy(g_hbm.at[sidx], rows, gsem).wait()
            pltpu.async_copy(rows, acc_ref.at[didx], ssem, add=True).wait()
            return 0

        lax.fori_loop(0, nch, body, 0)

    acc = jax.new_ref(jnp.concatenate([g, jnp.zeros((8, h), jnp.float32)]))
    k(g, src1d, dst1d, acc)
    return acc[...][:n]


def _dinv_call(parts):
    def body(p_ref, o_ref):
        deg = jnp.sum(p_ref[...], axis=0, keepdims=True)
        o_ref[...] = lax.rsqrt(deg + 1.0)

    return pl.pallas_call(
        body,
        out_shape=jax.ShapeDtypeStruct((1, parts.shape[1]), jnp.float32),
    )(parts)


def _mm_call(dinv, a, g, W, b):
    n, d = a.shape
    h = W.shape[1]
    bm = 2000 if n % 2000 == 0 else n
    grid = n // bm

    def body(dv_ref, a_ref, W_ref, b_ref, o_ref):
        av = a_ref[...]
        dv = dv_ref[...]
        if b is not None:
            av = dv * jnp.maximum(dv * av + b_ref[...], 0.0)
        else:
            av = dv * av
        o_ref[...] = lax.dot_general(av, W_ref[...], (((1,), (0,)), ((), ())),
                                     preferred_element_type=jnp.float32)

    bias = jnp.zeros((1, d), jnp.float32) if b is None else b.reshape(1, d)
    del g
    return pl.pallas_call(
        body,
        grid=(grid,),
        in_specs=[
            pl.BlockSpec((bm, 1), lambda i: (i, 0)),
            pl.BlockSpec((bm, d), lambda i: (i, 0)),
            pl.BlockSpec((d, h), lambda i: (0, 0)),
            pl.BlockSpec((1, d), lambda i: (0, 0)),
        ],
        out_specs=pl.BlockSpec((bm, h), lambda i: (i, 0)),
        out_shape=jax.ShapeDtypeStruct((n, h), jnp.float32),
    )(dinv, a, W, bias)


def _logits_call(dinv, a, Wa, b, ba):
    n, h = a.shape
    bm = 2000 if n % 2000 == 0 else n
    grid = n // bm

    def body(dv_ref, a_ref, Wa_ref, b_ref, ba_ref, o_ref):
        hv = jnp.maximum(dv_ref[...] * a_ref[...] + b_ref[...], 0.0)
        o_ref[...] = lax.dot_general(hv, Wa_ref[...], (((1,), (0,)), ((), ())),
                                     preferred_element_type=jnp.float32) + ba_ref[...]

    return pl.pallas_call(
        body,
        grid=(grid,),
        in_specs=[
            pl.BlockSpec((bm, 1), lambda i: (i, 0)),
            pl.BlockSpec((bm, h), lambda i: (i, 0)),
            pl.BlockSpec((h, 1), lambda i: (0, 0)),
            pl.BlockSpec((1, h), lambda i: (0, 0)),
            pl.BlockSpec((1, 1), lambda i: (0, 0)),
        ],
        out_specs=pl.BlockSpec((bm, 1), lambda i: (i, 0)),
        out_shape=jax.ShapeDtypeStruct((n, 1), jnp.float32),
    )(dinv, a, Wa, b.reshape(1, h), ba.reshape(1, 1))


def _softmax_call(logits, batch, maskf, num_seg):
    n = logits.shape[0]
    neg = -1e30

    def body(lg_ref, bt_ref, m_ref, o_ref):
        lg = lg_ref[...]
        bt = bt_ref[...]
        m = m_ref[...]
        masked = jnp.where(m > 0, lg, neg)
        seg = lax.broadcasted_iota(jnp.int32, (1, num_seg), 1)
        onehot = bt == seg
        vals = jnp.where(onehot, masked, neg)
        smax = jnp.max(vals, axis=0, keepdims=True)
        smax = jnp.where(smax > -1e29, smax, 0.0)
        pmax = jnp.sum(jnp.where(onehot, smax, 0.0), axis=1, keepdims=True)
        ex = jnp.where(m > 0, jnp.exp(masked - pmax), 0.0)
        den = jnp.sum(jnp.where(onehot, ex, 0.0), axis=0, keepdims=True)
        pden = jnp.sum(jnp.where(onehot, den, 0.0), axis=1, keepdims=True)
        o_ref[...] = ex / pden

    return pl.pallas_call(
        body,
        out_shape=jax.ShapeDtypeStruct((n, 1), jnp.float32),
    )(logits, batch, maskf)


def kernel(x, edge_index, mask, batch, W1, b1, W2, b2, W3, b3, Wa, ba):
    n, d = x.shape
    e = edge_index.shape[1]
    src = edge_index[0]
    dst = edge_index[1]

    t_pair = 2 * _cdiv(e, _NW * _CK)
    cpt0 = (3 * t_pair) // 5
    cpt1 = t_pair - cpt0
    e_pad = _NS * t_pair * _CK
    cpt = t_pair // 2
    src_p = jnp.concatenate([src, jnp.zeros((e_pad - e,), jnp.int32)])
    pad_i = jnp.arange(e, e_pad, dtype=jnp.int32)
    dst_p = jnp.concatenate([dst, n + (pad_i % 8)])
    npad = 16 * _cdiv(n + 1, 16)

    parts = _deg_call(dst_p, npad, cpt).reshape(_NW, npad)
    dinv = _dinv_call(parts).reshape(npad, 1)[:n]

    g1 = _mm_call(dinv, x, None, W1, None)
    a1 = _gs_call(g1, src_p, dst_p, cpt0, cpt1)
    g2 = _mm_call(dinv, a1, g1, W2, b1)
    a2 = _gs_call(g2, src_p, dst_p, cpt0, cpt1)
    g3 = _mm_call(dinv, a2, g2, W3, b2)
    a3 = _gs_call(g3, src_p, dst_p, cpt0, cpt1)

    logits = _logits_call(dinv, a3, Wa, b3, ba)
    probs = _softmax_call(logits, batch.reshape(n, 1),
                          mask.astype(jnp.float32).reshape(n, 1), 32)
    return probs.reshape(n)


# speedup vs baseline: 1.1409x; 1.0105x over previous
import functools

import jax
import jax.numpy as jnp
from jax import lax
from jax.experimental import pallas as pl
from jax.experimental.pallas import tpu as pltpu
from jax.experimental.pallas import tpu_sc as plsc

_NS = 16
_NC = 2
_NW = _NC * _NS
_CK = 128


def _cdiv(a, b):
    return (a + b - 1) // b


def _deg_call(dst1d, npad, cpt):
    mesh = plsc.VectorSubcoreMesh(core_axis_name="c", subcore_axis_name="s")

    @functools.partial(
        pl.kernel,
        mesh=mesh,
        out_type=jax.ShapeDtypeStruct((_NW * npad,), jnp.float32),
        compiler_params=pltpu.CompilerParams(needs_layout_passes=False),
        scratch_types=[
            pltpu.VMEM((_CK,), jnp.int32),
            pltpu.VMEM((npad,), jnp.float32),
        ],
    )
    def k(dst_hbm, out_hbm, didx, priv):
        c = lax.axis_index("c")
        s = lax.axis_index("s")
        wid = c * _NS + s

        def zero(i, _):
            priv[pl.ds(i * 16, 16)] = jnp.zeros((16,), jnp.float32)
            return 0

        lax.fori_loop(0, npad // 16, zero, 0)
        ones16 = jnp.ones((16,), jnp.float32)

        def body(j, _):
            off = (wid * cpt + j) * _CK
            pltpu.sync_copy(dst_hbm.at[pl.ds(off, _CK)], didx)
            for i in range(_CK // 16):
                idx = didx[pl.ds(i * 16, 16)]
                plsc.addupdate_scatter(priv, [idx], ones16)
            return 0

        lax.fori_loop(0, cpt, body, 0)
        pltpu.sync_copy(priv, out_hbm.at[pl.ds(wid * npad, npad)])

    return k(dst1d)


_CKL = 128


def _gs_call(g, src1d, dst1d, cpt0, cpt1):
    n, h = g.shape
    mesh = plsc.VectorSubcoreMesh(core_axis_name="c", subcore_axis_name="s")

    @functools.partial(
        pl.kernel,
        mesh=mesh,
        out_type=(),
        scratch_types=[
            pltpu.VMEM((_CKL,), jnp.int32),
            pltpu.VMEM((_CKL,), jnp.int32),
            pltpu.VMEM((_CKL, h), jnp.float32),
            pltpu.SemaphoreType.DMA,
            pltpu.SemaphoreType.DMA,
        ],
    )
    def k(g_hbm, src_hbm, dst_hbm, acc_ref, sidx, didx, rows, gsem, ssem):
        c = lax.axis_index("c")
        s = lax.axis_index("s")
        base = jnp.where(c == 0, s * cpt0,
                         _NS * cpt0 + s * cpt1) * _CKL
        nch = jnp.where(c == 0, cpt0, cpt1)

        def body(j, _):
            off = base + j * _CKL
            pltpu.sync_copy(src_hbm.at[pl.ds(off, _CKL)], sidx)
            pltpu.sync_copy(dst_hbm.at[pl.ds(off, _CKL)], didx)
            pltpu.async_copy(g_hbm.at[sidx], rows, gsem).wait()
            pltpu.async_copy(rows, acc_ref.at[didx], ssem, add=True).wait()
            return 0

        lax.fori_loop(0, nch, body, 0)

    acc = jax.new_ref(jnp.concatenate([g, jnp.zeros((8, h), jnp.float32)]))
    k(g, src1d, dst1d, acc)
    return acc[...][:n]


def _dinv_call(parts):
    def body(p_ref, o_ref):
        deg = jnp.sum(p_ref[...], axis=0, keepdims=True)
        o_ref[...] = lax.rsqrt(deg + 1.0)

    return pl.pallas_call(
        body,
        out_shape=jax.ShapeDtypeStruct((1, parts.shape[1]), jnp.float32),
    )(parts)


def _mm_call(dinv, a, g, W, b):
    n, d = a.shape
    h = W.shape[1]
    bm = 2000 if n % 2000 == 0 else n
    grid = n // bm

    def body(dv_ref, a_ref, W_ref, b_ref, o_ref):
        av = a_ref[...]
        dv = dv_ref[...]
        if b is not None:
            av = dv * jnp.maximum(dv * av + b_ref[...], 0.0)
        else:
            av = dv * av
        o_ref[...] = lax.dot_general(av, W_ref[...], (((1,), (0,)), ((), ())),
                                     preferred_element_type=jnp.float32)

    bias = jnp.zeros((1, d), jnp.float32) if b is None else b.reshape(1, d)
    del g
    return pl.pallas_call(
        body,
        grid=(grid,),
        in_specs=[
            pl.BlockSpec((bm, 1), lambda i: (i, 0)),
            pl.BlockSpec((bm, d), lambda i: (i, 0)),
            pl.BlockSpec((d, h), lambda i: (0, 0)),
            pl.BlockSpec((1, d), lambda i: (0, 0)),
        ],
        out_specs=pl.BlockSpec((bm, h), lambda i: (i, 0)),
        out_shape=jax.ShapeDtypeStruct((n, h), jnp.float32),
    )(dinv, a, W, bias)


def _logits_call(dinv, a, Wa, b, ba):
    n, h = a.shape
    bm = 2000 if n % 2000 == 0 else n
    grid = n // bm

    def body(dv_ref, a_ref, Wa_ref, b_ref, ba_ref, o_ref):
        hv = jnp.maximum(dv_ref[...] * a_ref[...] + b_ref[...], 0.0)
        o_ref[...] = lax.dot_general(hv, Wa_ref[...], (((1,), (0,)), ((), ())),
                                     preferred_element_type=jnp.float32) + ba_ref[...]

    return pl.pallas_call(
        body,
        grid=(grid,),
        in_specs=[
            pl.BlockSpec((bm, 1), lambda i: (i, 0)),
            pl.BlockSpec((bm, h), lambda i: (i, 0)),
            pl.BlockSpec((h, 1), lambda i: (0, 0)),
            pl.BlockSpec((1, h), lambda i: (0, 0)),
            pl.BlockSpec((1, 1), lambda i: (0, 0)),
        ],
        out_specs=pl.BlockSpec((bm, 1), lambda i: (i, 0)),
        out_shape=jax.ShapeDtypeStruct((n, 1), jnp.float32),
    )(dinv, a, Wa, b.reshape(1, h), ba.reshape(1, 1))


def _softmax_call(logits, batch, maskf, num_seg):
    n = logits.shape[0]
    neg = -1e30

    def body(lg_ref, bt_ref, m_ref, o_ref):
        lg = lg_ref[...]
        bt = bt_ref[...]
        m = m_ref[...]
        masked = jnp.where(m > 0, lg, neg)
        seg = lax.broadcasted_iota(jnp.int32, (1, num_seg), 1)
        onehot = bt == seg
        vals = jnp.where(onehot, masked, neg)
        smax = jnp.max(vals, axis=0, keepdims=True)
        smax = jnp.where(smax > -1e29, smax, 0.0)
        pmax = jnp.sum(jnp.where(onehot, smax, 0.0), axis=1, keepdims=True)
        ex = jnp.where(m > 0, jnp.exp(masked - pmax), 0.0)
        den = jnp.sum(jnp.where(onehot, ex, 0.0), axis=0, keepdims=True)
        pden = jnp.sum(jnp.where(onehot, den, 0.0), axis=1, keepdims=True)
        o_ref[...] = ex / pden

    return pl.pallas_call(
        body,
        out_shape=jax.ShapeDtypeStruct((n, 1), jnp.float32),
    )(logits, batch, maskf)


def kernel(x, edge_index, mask, batch, W1, b1, W2, b2, W3, b3, Wa, ba):
    n, d = x.shape
    e = edge_index.shape[1]
    src = edge_index[0]
    dst = edge_index[1]

    t_pair = 2 * _cdiv(e, _NW * _CK)
    cpt0 = (63 * t_pair) // 100
    cpt1 = t_pair - cpt0
    e_pad = _NS * t_pair * _CK
    cpt = t_pair // 2
    src_p = jnp.concatenate([src, jnp.zeros((e_pad - e,), jnp.int32)])
    pad_i = jnp.arange(e, e_pad, dtype=jnp.int32)
    dst_p = jnp.concatenate([dst, n + (pad_i % 8)])
    npad = 16 * _cdiv(n + 1, 16)

    parts = _deg_call(dst_p, npad, cpt).reshape(_NW, npad)
    dinv = _dinv_call(parts).reshape(npad, 1)[:n]

    g1 = _mm_call(dinv, x, None, W1, None)
    a1 = _gs_call(g1, src_p, dst_p, cpt0, cpt1)
    g2 = _mm_call(dinv, a1, g1, W2, b1)
    a2 = _gs_call(g2, src_p, dst_p, cpt0, cpt1)
    g3 = _mm_call(dinv, a2, g2, W3, b2)
    a3 = _gs_call(g3, src_p, dst_p, cpt0, cpt1)

    logits = _logits_call(dinv, a3, Wa, b3, ba)
    probs = _softmax_call(logits, batch.reshape(n, 1),
                          mask.astype(jnp.float32).reshape(n, 1), 32)
    return probs.reshape(n)


# speedup vs baseline: 1.1464x; 1.0048x over previous
import functools

import jax
import jax.numpy as jnp
from jax import lax
from jax.experimental import pallas as pl
from jax.experimental.pallas import tpu as pltpu
from jax.experimental.pallas import tpu_sc as plsc

_NS = 16
_NC = 2
_NW = _NC * _NS
_CK = 128


def _cdiv(a, b):
    return (a + b - 1) // b


def _deg_call(dst1d, npad, cpt):
    mesh = plsc.VectorSubcoreMesh(core_axis_name="c", subcore_axis_name="s")

    @functools.partial(
        pl.kernel,
        mesh=mesh,
        out_type=jax.ShapeDtypeStruct((_NW * npad,), jnp.float32),
        compiler_params=pltpu.CompilerParams(needs_layout_passes=False),
        scratch_types=[
            pltpu.VMEM((_CK,), jnp.int32),
            pltpu.VMEM((npad,), jnp.float32),
        ],
    )
    def k(dst_hbm, out_hbm, didx, priv):
        c = lax.axis_index("c")
        s = lax.axis_index("s")
        wid = c * _NS + s

        def zero(i, _):
            priv[pl.ds(i * 16, 16)] = jnp.zeros((16,), jnp.float32)
            return 0

        lax.fori_loop(0, npad // 16, zero, 0)
        ones16 = jnp.ones((16,), jnp.float32)

        def body(j, _):
            off = (wid * cpt + j) * _CK
            pltpu.sync_copy(dst_hbm.at[pl.ds(off, _CK)], didx)
            for i in range(_CK // 16):
                idx = didx[pl.ds(i * 16, 16)]
                plsc.addupdate_scatter(priv, [idx], ones16)
            return 0

        lax.fori_loop(0, cpt, body, 0)
        pltpu.sync_copy(priv, out_hbm.at[pl.ds(wid * npad, npad)])

    return k(dst1d)


_CKL = 128


def _gs_call(g, src1d, dst1d, cpt0, cpt1):
    n, h = g.shape
    mesh = plsc.VectorSubcoreMesh(core_axis_name="c", subcore_axis_name="s")

    @functools.partial(
        pl.kernel,
        mesh=mesh,
        out_type=(),
        scratch_types=[
            pltpu.VMEM((_CKL,), jnp.int32),
            pltpu.VMEM((_CKL,), jnp.int32),
            pltpu.VMEM((_CKL, h), jnp.float32),
            pltpu.SemaphoreType.DMA,
            pltpu.SemaphoreType.DMA,
        ],
    )
    def k(g_hbm, src_hbm, dst_hbm, acc_ref, sidx, didx, rows, gsem, ssem):
        c = lax.axis_index("c")
        s = lax.axis_index("s")
        base = jnp.where(c == 0, s * cpt0,
                         _NS * cpt0 + s * cpt1) * _CKL
        nch = jnp.where(c == 0, cpt0, cpt1)

        def body(j, _):
            off = base + j * _CKL
            pltpu.sync_copy(src_hbm.at[pl.ds(off, _CKL)], sidx)
            pltpu.sync_copy(dst_hbm.at[pl.ds(off, _CKL)], didx)
            pltpu.async_copy(g_hbm.at[sidx], rows, gsem).wait()
            pltpu.async_copy(rows, acc_ref.at[didx], ssem, add=True).wait()
            return 0

        lax.fori_loop(0, nch, body, 0)

    acc = jax.new_ref(jnp.concatenate([g, jnp.zeros((8, h), jnp.float32)]))
    k(g, src1d, dst1d, acc)
    return acc[...][:n]


def _dinv_call(parts):
    def body(p_ref, o_ref):
        deg = jnp.sum(p_ref[...], axis=0, keepdims=True)
        o_ref[...] = lax.rsqrt(deg + 1.0)

    return pl.pallas_call(
        body,
        out_shape=jax.ShapeDtypeStruct((1, parts.shape[1]), jnp.float32),
    )(parts)


def _mm_call(dinv, a, g, W, b):
    n, d = a.shape
    h = W.shape[1]
    bm = 2000 if n % 2000 == 0 else n
    grid = n // bm

    def body(dv_ref, a_ref, W_ref, b_ref, o_ref):
        av = a_ref[...]
        dv = dv_ref[...]
        if b is not None:
            av = dv * jnp.maximum(dv * av + b_ref[...], 0.0)
        else:
            av = dv * av
        o_ref[...] = lax.dot_general(av, W_ref[...], (((1,), (0,)), ((), ())),
                                     preferred_element_type=jnp.float32)

    bias = jnp.zeros((1, d), jnp.float32) if b is None else b.reshape(1, d)
    del g
    return pl.pallas_call(
        body,
        grid=(grid,),
        in_specs=[
            pl.BlockSpec((bm, 1), lambda i: (i, 0)),
            pl.BlockSpec((bm, d), lambda i: (i, 0)),
            pl.BlockSpec((d, h), lambda i: (0, 0)),
            pl.BlockSpec((1, d), lambda i: (0, 0)),
        ],
        out_specs=pl.BlockSpec((bm, h), lambda i: (i, 0)),
        out_shape=jax.ShapeDtypeStruct((n, h), jnp.float32),
    )(dinv, a, W, bias)


def _head_call(dinv, a, Wa, b, ba, batch, maskf, num_seg):
    n, h = a.shape
    neg = -1e30

    def body(dv_ref, a_ref, Wa_ref, b_ref, ba_ref, bt_ref, m_ref, o_ref):
        hv = jnp.maximum(dv_ref[...] * a_ref[...] + b_ref[...], 0.0)
        lg = lax.dot_general(hv, Wa_ref[...], (((1,), (0,)), ((), ())),
                             preferred_element_type=jnp.float32) + ba_ref[...]
        bt = bt_ref[...]
        m = m_ref[...]
        masked = jnp.where(m > 0, lg, neg)
        seg = lax.broadcasted_iota(jnp.int32, (1, num_seg), 1)
        onehot = bt == seg
        vals = jnp.where(onehot, masked, neg)
        smax = jnp.max(vals, axis=0, keepdims=True)
        smax = jnp.where(smax > -1e29, smax, 0.0)
        pmax = jnp.sum(jnp.where(onehot, smax, 0.0), axis=1, keepdims=True)
        ex = jnp.where(m > 0, jnp.exp(masked - pmax), 0.0)
        den = jnp.sum(jnp.where(onehot, ex, 0.0), axis=0, keepdims=True)
        pden = jnp.sum(jnp.where(onehot, den, 0.0), axis=1, keepdims=True)
        o_ref[...] = ex / pden

    return pl.pallas_call(
        body,
        out_shape=jax.ShapeDtypeStruct((n, 1), jnp.float32),
    )(dinv, a, Wa, b.reshape(1, h), ba.reshape(1, 1), batch, maskf)


def kernel(x, edge_index, mask, batch, W1, b1, W2, b2, W3, b3, Wa, ba):
    n, d = x.shape
    e = edge_index.shape[1]
    src = edge_index[0]
    dst = edge_index[1]

    t_pair = 2 * _cdiv(e, _NW * _CK)
    cpt0 = (63 * t_pair) // 100
    cpt1 = t_pair - cpt0
    e_pad = _NS * t_pair * _CK
    cpt = t_pair // 2
    src_p = jnp.concatenate([src, jnp.zeros((e_pad - e,), jnp.int32)])
    pad_i = jnp.arange(e, e_pad, dtype=jnp.int32)
    dst_p = jnp.concatenate([dst, n + (pad_i % 8)])
    npad = 16 * _cdiv(n + 1, 16)

    parts = _deg_call(dst_p, npad, cpt).reshape(_NW, npad)
    dinv = _dinv_call(parts).reshape(npad, 1)[:n]

    g1 = _mm_call(dinv, x, None, W1, None)
    a1 = _gs_call(g1, src_p, dst_p, cpt0, cpt1)
    g2 = _mm_call(dinv, a1, g1, W2, b1)
    a2 = _gs_call(g2, src_p, dst_p, cpt0, cpt1)
    g3 = _mm_call(dinv, a2, g2, W3, b2)
    a3 = _gs_call(g3, src_p, dst_p, cpt0, cpt1)

    probs = _head_call(dinv, a3, Wa, b3, ba, batch.reshape(n, 1),
                       mask.astype(jnp.float32).reshape(n, 1), 32)
    return probs.reshape(n)
